# baseline (speedup 1.0000x reference)
import functools

import jax
import jax.numpy as jnp
from jax import lax
from jax.experimental import pallas as pl
from jax.experimental.pallas import tpu as pltpu
from jax.experimental.pallas import tpu_sc as plsc

NC = 2
NS = 16
NW = NC * NS


NQ = 20


def _agg_body(iq_hbm, table_hbm, zn_hbm, parts_out,
              idxq, rows_v, sem, acc):
    cid = lax.axis_index("c")
    sid = lax.axis_index("s")
    w = cid * NS + sid
    n_rows = acc.shape[0]
    stripe = (n_rows // NS) & ~7
    rem = n_rows - stripe * NS

    pltpu.sync_copy(zn_hbm.at[pl.ds(0, stripe)],
                    acc.at[pl.ds(sid * stripe, stripe)])
    if rem:
        @pl.when(sid == NS - 1)
        def _():
            pltpu.sync_copy(zn_hbm.at[pl.ds(stripe, rem)],
                            acc.at[pl.ds(stripe * NS, rem)])
    plsc.subcore_barrier()

    def quad(q, _):
        qoff = pl.multiple_of((w * NQ + q) * 8, 8)
        pltpu.sync_copy(iq_hbm.at[pl.ds(qoff, 8)], idxq)
        for k in range(4):
            pltpu.async_copy(table_hbm.at[idxq.at[k]], rows_v, sem).wait()
            pltpu.sync_copy(rows_v, acc.at[idxq.at[4 + k]], add=True)
        return 0

    lax.fori_loop(0, NQ, quad, 0)

    plsc.subcore_barrier()
    poff = pl.multiple_of(cid * n_rows + sid * stripe, 8)
    pltpu.sync_copy(acc.at[pl.ds(sid * stripe, stripe)],
                    parts_out.at[pl.ds(poff, stripe)])
    if rem:
        @pl.when(sid == NS - 1)
        def _():
            poff2 = pl.multiple_of(cid * n_rows + stripe * NS, 8)
            pltpu.sync_copy(acc.at[pl.ds(stripe * NS, rem)],
                            parts_out.at[pl.ds(poff2, rem)])


def _gidx_tc(n_nodes, per_w, et_ref, src_ref, g_ref, d2_ref):
    et = et_ref[...]
    shape = et.shape
    pos = (lax.broadcasted_iota(jnp.int32, shape, 0) * shape[1]
           + lax.broadcasted_iota(jnp.int32, shape, 1))
    g_ref[...] = et * n_nodes + src_ref[...]
    d2_ref[...] = et + 8 * ((pos // per_w) % NS)


def _wmix_tc(comp_ref, vflat_ref, o_ref):
    o_ref[...] = jnp.dot(comp_ref[...], vflat_ref[...],
                         preferred_element_type=jnp.float32)


def _proj_tc(x_ref, w_ref, o_ref):
    o_ref[0] = jnp.dot(x_ref[...], w_ref[0],
                       preferred_element_type=jnp.float32)


def _combine_tc(p_ref, b_ref, o_ref):
    o_ref[...] = jax.nn.relu(p_ref[0] + p_ref[1] + b_ref[...])


def _treduce_tc(p_ref, o_ref):
    v = p_ref[...]
    acc = v[0, 0:8]
    for g in range(NS):
        for c in range(NC):
            if c == 0 and g == 0:
                continue
            acc = acc + v[c, 8 * g:8 * g + 8]
    o_ref[...] = acc


def _final_tc(n_nodes, t_ref, comp2t_ref, v2flat_ref, b2_ref, fcw_ref,
              fcb_ref, o_ref):
    u = jnp.dot(comp2t_ref[...], t_ref[...],
                preferred_element_type=jnp.float32)
    s = jnp.dot(u.reshape(1, -1), v2flat_ref[...],
                preferred_element_type=jnp.float32)
    hg = s * (1.0 / n_nodes) + b2_ref[...]
    o_ref[...] = jnp.dot(hg, fcw_ref[...],
                         preferred_element_type=jnp.float32) + fcb_ref[...]


def kernel(features, edge_index, etypes, V1, comp1, b1, V2, comp2, b2,
           fcW, fcb):
    n, din = features.shape
    e = etypes.shape[0]
    r = V1.shape[0]
    h = V1.shape[2]
    out_d = V2.shape[2]
    f32 = jnp.float32

    src = edge_index[0]
    dst = edge_index[1]

    per_w = e // NW
    assert per_w * NW == e and n % NS == 0
    bins = r * n

    mesh = plsc.VectorSubcoreMesh(core_axis_name="c", subcore_axis_name="s",
                                  num_cores=NC, num_subcores=NS)

    n_acc1 = n + 16
    zn_stripe = (n_acc1 // NS) & ~7
    zn = jnp.zeros((zn_stripe + (n_acc1 - zn_stripe * NS), h), f32)

    def pack_quads(gidx_a, dst_a, dst_fill):
        g2 = jnp.pad(gidx_a.reshape(NW, per_w),
                     ((0, 0), (0, NQ * 512 - per_w)), constant_values=0)
        d2 = jnp.pad(dst_a.reshape(NW, per_w),
                     ((0, 0), (0, NQ * 512 - per_w)),
                     constant_values=dst_fill)
        iq = jnp.concatenate([g2.reshape(NW, NQ, 4, 128),
                              d2.reshape(NW, NQ, 4, 128)], axis=2)
        return iq.reshape(NW * NQ * 8, 128)

    def sc_agg(iq, table, acc_rows):
        return pl.kernel(
            _agg_body,
            out_type=jax.ShapeDtypeStruct((NC * acc_rows, h), f32),
            mesh=mesh,
            scratch_types=(
                pltpu.VMEM((8, 128), jnp.int32),
                pltpu.VMEM((128, h), f32),
                pltpu.SemaphoreType.DMA,
                pltpu.VMEM_SHARED((acc_rows, h), f32),
            ),
        )(iq, table, zn)

    erows = 320
    ecols = e // erows
    gidx, dst2 = pl.pallas_call(
        functools.partial(_gidx_tc, n, per_w),
        grid=(1,),
        in_specs=[pl.BlockSpec((erows, ecols), lambda j: (0, 0)),
                  pl.BlockSpec((erows, ecols), lambda j: (0, 0))],
        out_specs=[pl.BlockSpec((erows, ecols), lambda j: (0, 0)),
                   pl.BlockSpec((erows, ecols), lambda j: (0, 0))],
        out_shape=(jax.ShapeDtypeStruct((erows, ecols), jnp.int32),
                   jax.ShapeDtypeStruct((erows, ecols), jnp.int32)),
    )(etypes.reshape(erows, ecols), src.reshape(erows, ecols))
    gidx = gidx.reshape(e)
    dst2 = dst2.reshape(e)

    w1 = pl.pallas_call(
        _wmix_tc,
        out_shape=jax.ShapeDtypeStruct((r, din * h), f32),
    )(comp1, V1.reshape(r, din * h))

    nblk = 1000
    hs1 = pl.pallas_call(
        _proj_tc,
        grid=(r, n // nblk),
        in_specs=[
            pl.BlockSpec((nblk, din), lambda ri, j: (j, 0)),
            pl.BlockSpec((1, din, h), lambda ri, j: (ri, 0, 0)),
        ],
        out_specs=pl.BlockSpec((1, nblk, h), lambda ri, j: (ri, j, 0)),
        out_shape=jax.ShapeDtypeStruct((r, n, h), f32),
    )(features, w1.reshape(r, din, h))

    iq1 = pack_quads(gidx.reshape(e), dst, n)
    parts = sc_agg(iq1, hs1.reshape(bins, h),
                   n_acc1).reshape(NC, n_acc1, h)

    h1 = pl.pallas_call(
        _combine_tc,
        grid=(n // nblk,),
        in_specs=[
            pl.BlockSpec((NC, nblk, h), lambda j: (0, j, 0)),
            pl.BlockSpec((1, h), lambda j: (0, 0)),
        ],
        out_specs=pl.BlockSpec((nblk, h), lambda j: (j, 0)),
        out_shape=jax.ShapeDtypeStruct((n, h), f32),
    )(parts, b1.reshape(1, h))

    iq2 = pack_quads(src, dst2.reshape(e), 8 * NS)
    parts2 = sc_agg(iq2, h1, 8 * NS + 16).reshape(NC, 8 * NS + 16, h)

    t = pl.pallas_call(
        _treduce_tc,
        grid=(1,),
        in_specs=[pl.BlockSpec((NC, 8 * NS, h), lambda j: (0, 0, 0))],
        out_specs=pl.BlockSpec((r, h), lambda j: (0, 0)),
        out_shape=jax.ShapeDtypeStruct((r, h), f32),
    )(parts2)

    out = pl.pallas_call(
        functools.partial(_final_tc, n),
        out_shape=jax.ShapeDtypeStruct((1, fcW.shape[1]), f32),
    )(t, comp2.T, V2.reshape(r * h, out_d), b2.reshape(1, out_d),
      fcW, fcb.reshape(1, fcW.shape[1]))

    return out


# speedup vs baseline: 1.9715x; 1.9715x over previous
import functools

import jax
import jax.numpy as jnp
from jax import lax
from jax.experimental import pallas as pl
from jax.experimental.pallas import tpu as pltpu
from jax.experimental.pallas import tpu_sc as plsc

NC = 2
NS = 16
NW = NC * NS


def _agg_body(per_w, gidx_hbm, dst_hbm, table_hbm, zn_hbm, parts_out,
              idx_v, didx_v, idx16, didx16, rows_v, rows16, sem, acc):
    cid = lax.axis_index("c")
    sid = lax.axis_index("s")
    w = cid * NS + sid
    base = w * per_w
    n_rows = acc.shape[0]
    stripe = (n_rows // NS) & ~7
    rem = n_rows - stripe * NS

    pltpu.sync_copy(zn_hbm.at[pl.ds(0, stripe)],
                    acc.at[pl.ds(sid * stripe, stripe)])
    if rem:
        @pl.when(sid == NS - 1)
        def _():
            pltpu.sync_copy(zn_hbm.at[pl.ds(stripe, rem)],
                            acc.at[pl.ds(stripe * NS, rem)])
    plsc.subcore_barrier()

    nfull = per_w // 128

    def chunk(i, _):
        off = pl.multiple_of(base + i * 128, 16)
        pltpu.sync_copy(gidx_hbm.at[pl.ds(off, 128)], idx_v)
        da = pltpu.async_copy(table_hbm.at[idx_v], rows_v, sem)
        pltpu.sync_copy(dst_hbm.at[pl.ds(off, 128)], didx_v)
        da.wait()
        pltpu.sync_copy(rows_v, acc.at[didx_v], add=True)
        return 0

    lax.fori_loop(0, nfull, chunk, 0)

    tail = per_w - nfull * 128
    if tail:
        off = pl.multiple_of(base + nfull * 128, 16)
        pltpu.sync_copy(gidx_hbm.at[pl.ds(off, tail)], idx16)
        dt = pltpu.async_copy(table_hbm.at[idx16], rows16, sem)
        pltpu.sync_copy(dst_hbm.at[pl.ds(off, tail)], didx16)
        dt.wait()
        pltpu.sync_copy(rows16, acc.at[didx16], add=True)

    plsc.subcore_barrier()
    poff = pl.multiple_of(cid * n_rows + sid * stripe, 8)
    pltpu.sync_copy(acc.at[pl.ds(sid * stripe, stripe)],
                    parts_out.at[pl.ds(poff, stripe)])
    if rem:
        @pl.when(sid == NS - 1)
        def _():
            poff2 = pl.multiple_of(cid * n_rows + stripe * NS, 8)
            pltpu.sync_copy(acc.at[pl.ds(stripe * NS, rem)],
                            parts_out.at[pl.ds(poff2, rem)])


def _gidx_tc(n_nodes, per_w, et_ref, src_ref, g_ref, d2_ref):
    et = et_ref[...]
    shape = et.shape
    pos = (lax.broadcasted_iota(jnp.int32, shape, 0) * shape[1]
           + lax.broadcasted_iota(jnp.int32, shape, 1))
    g_ref[...] = et * n_nodes + src_ref[...]
    d2_ref[...] = et + 8 * ((pos // per_w) % NS)


def _wmix_tc(comp_ref, vflat_ref, o_ref):
    o_ref[...] = jnp.dot(comp_ref[...], vflat_ref[...],
                         preferred_element_type=jnp.float32)


def _proj_tc(x_ref, w_ref, o_ref):
    o_ref[0] = jnp.dot(x_ref[...], w_ref[0],
                       preferred_element_type=jnp.float32)


def _combine_tc(p_ref, b_ref, o_ref):
    o_ref[...] = jax.nn.relu(p_ref[0] + p_ref[1] + b_ref[...])


def _treduce_tc(p_ref, o_ref):
    v = p_ref[...]
    acc = v[0, 0:8]
    for g in range(NS):
        for c in range(NC):
            if c == 0 and g == 0:
                continue
            acc = acc + v[c, 8 * g:8 * g + 8]
    o_ref[...] = acc


def _final_tc(n_nodes, t_ref, comp2t_ref, v2flat_ref, b2_ref, fcw_ref,
              fcb_ref, o_ref):
    u = jnp.dot(comp2t_ref[...], t_ref[...],
                preferred_element_type=jnp.float32)
    s = jnp.dot(u.reshape(1, -1), v2flat_ref[...],
                preferred_element_type=jnp.float32)
    hg = s * (1.0 / n_nodes) + b2_ref[...]
    o_ref[...] = jnp.dot(hg, fcw_ref[...],
                         preferred_element_type=jnp.float32) + fcb_ref[...]


def kernel(features, edge_index, etypes, V1, comp1, b1, V2, comp2, b2,
           fcW, fcb):
    n, din = features.shape
    e = etypes.shape[0]
    r = V1.shape[0]
    h = V1.shape[2]
    out_d = V2.shape[2]
    f32 = jnp.float32

    src = edge_index[0]
    dst = edge_index[1]

    per_w = e // NW
    assert per_w * NW == e and n % NS == 0
    bins = r * n

    mesh = plsc.VectorSubcoreMesh(core_axis_name="c", subcore_axis_name="s",
                                  num_cores=NC, num_subcores=NS)

    zn_stripe = (n // NS) & ~7
    zn = jnp.zeros((zn_stripe + (n - zn_stripe * NS), h), f32)

    def sc_agg(gidx_a, dst_a, table, acc_rows):
        return pl.kernel(
            functools.partial(_agg_body, per_w),
            out_type=jax.ShapeDtypeStruct((NC * acc_rows, h), f32),
            mesh=mesh,
            scratch_types=(
                pltpu.VMEM((128,), jnp.int32),
                pltpu.VMEM((128,), jnp.int32),
                pltpu.VMEM((16,), jnp.int32),
                pltpu.VMEM((16,), jnp.int32),
                pltpu.VMEM((128, h), f32),
                pltpu.VMEM((16, h), f32),
                pltpu.SemaphoreType.DMA,
                pltpu.VMEM_SHARED((acc_rows, h), f32),
            ),
        )(gidx_a, dst_a, table, zn)

    erows = 320
    ecols = e // erows
    gidx, dst2 = pl.pallas_call(
        functools.partial(_gidx_tc, n, per_w),
        grid=(1,),
        in_specs=[pl.BlockSpec((erows, ecols), lambda j: (0, 0)),
                  pl.BlockSpec((erows, ecols), lambda j: (0, 0))],
        out_specs=[pl.BlockSpec((erows, ecols), lambda j: (0, 0)),
                   pl.BlockSpec((erows, ecols), lambda j: (0, 0))],
        out_shape=(jax.ShapeDtypeStruct((erows, ecols), jnp.int32),
                   jax.ShapeDtypeStruct((erows, ecols), jnp.int32)),
    )(etypes.reshape(erows, ecols), src.reshape(erows, ecols))
    gidx = gidx.reshape(e)
    dst2 = dst2.reshape(e)

    w1 = pl.pallas_call(
        _wmix_tc,
        out_shape=jax.ShapeDtypeStruct((r, din * h), f32),
    )(comp1, V1.reshape(r, din * h))

    nblk = 1000
    hs1 = pl.pallas_call(
        _proj_tc,
        grid=(r, n // nblk),
        in_specs=[
            pl.BlockSpec((nblk, din), lambda ri, j: (j, 0)),
            pl.BlockSpec((1, din, h), lambda ri, j: (ri, 0, 0)),
        ],
        out_specs=pl.BlockSpec((1, nblk, h), lambda ri, j: (ri, j, 0)),
        out_shape=jax.ShapeDtypeStruct((r, n, h), f32),
    )(features, w1.reshape(r, din, h))

    parts = sc_agg(gidx, dst, hs1.reshape(bins, h), n).reshape(NC, n, h)

    h1 = pl.pallas_call(
        _combine_tc,
        grid=(n // nblk,),
        in_specs=[
            pl.BlockSpec((NC, nblk, h), lambda j: (0, j, 0)),
            pl.BlockSpec((1, h), lambda j: (0, 0)),
        ],
        out_specs=pl.BlockSpec((nblk, h), lambda j: (j, 0)),
        out_shape=jax.ShapeDtypeStruct((n, h), f32),
    )(parts, b1.reshape(1, h))

    parts2 = sc_agg(src, dst2, h1, 8 * NS).reshape(NC, 8 * NS, h)

    t = pl.pallas_call(
        _treduce_tc,
        grid=(1,),
        in_specs=[pl.BlockSpec((NC, 8 * NS, h), lambda j: (0, 0, 0))],
        out_specs=pl.BlockSpec((r, h), lambda j: (0, 0)),
        out_shape=jax.ShapeDtypeStruct((r, h), f32),
    )(parts2)

    out = pl.pallas_call(
        functools.partial(_final_tc, n),
        out_shape=jax.ShapeDtypeStruct((1, fcW.shape[1]), f32),
    )(t, comp2.T, V2.reshape(r * h, out_d), b2.reshape(1, out_d),
      fcW, fcb.reshape(1, fcW.shape[1]))

    return out
